# Initial kernel scaffold; baseline (speedup 1.0000x reference)
import jax
import jax.numpy as jnp
from jax.experimental import pallas as pl


def kernel(hidden_states, Wg, W1, W3, W2):
    raise NotImplementedError("write your pallas kernel here")


# speedup vs baseline: 1.0376x; 1.0376x over previous
import functools

import jax
import jax.numpy as jnp
from jax.experimental import pallas as pl
from jax.experimental.pallas import tpu as pltpu

_BT = 256


def _moe_dense_body(x_ref, wg_ref, w1_ref, w3_ref, w2_ref, out_ref):
    e = pl.program_id(1)
    x = x_ref[...]
    logits = jnp.dot(x, wg_ref[...], preferred_element_type=jnp.float32)
    probs = jax.nn.softmax(logits, axis=-1)
    m1 = jnp.max(probs, axis=-1)
    i1 = jnp.argmax(probs, axis=-1)
    lane = jax.lax.broadcasted_iota(jnp.int32, probs.shape, 1)
    probs2 = jnp.where(lane == i1[:, None], -jnp.inf, probs)
    m2 = jnp.max(probs2, axis=-1)
    i2 = jnp.argmax(probs2, axis=-1)
    w = jnp.where(e == i1, m1, jnp.where(e == i2, m2, 0.0)) / (m1 + m2)
    g = jnp.dot(x, w1_ref[0], preferred_element_type=jnp.float32)
    u = jnp.dot(x, w3_ref[0], preferred_element_type=jnp.float32)
    h = (g * jax.nn.sigmoid(g)) * u
    oe = jnp.dot(h, w2_ref[0], preferred_element_type=jnp.float32)
    contrib = oe * w[:, None]

    @pl.when(e == 0)
    def _():
        out_ref[...] = contrib

    @pl.when(e != 0)
    def _():
        out_ref[...] += contrib


def kernel(hidden_states, Wg, W1, W3, W2):
    T, D = hidden_states.shape
    E = Wg.shape[1]
    F = W1.shape[2]
    grid = (T // _BT, E)
    return pl.pallas_call(
        _moe_dense_body,
        grid=grid,
        in_specs=[
            pl.BlockSpec((_BT, D), lambda t, e: (t, 0)),
            pl.BlockSpec((D, E), lambda t, e: (0, 0)),
            pl.BlockSpec((1, D, F), lambda t, e: (e, 0, 0)),
            pl.BlockSpec((1, D, F), lambda t, e: (e, 0, 0)),
            pl.BlockSpec((1, F, D), lambda t, e: (e, 0, 0)),
        ],
        out_specs=pl.BlockSpec((_BT, D), lambda t, e: (t, 0)),
        out_shape=jax.ShapeDtypeStruct((T, D), jnp.float32),
        compiler_params=pltpu.CompilerParams(
            dimension_semantics=("parallel", "arbitrary"),
        ),
    )(hidden_states, Wg, W1, W3, W2)


# speedup vs baseline: 1.1943x; 1.1510x over previous
import functools

import jax
import jax.numpy as jnp
from jax.experimental import pallas as pl
from jax.experimental.pallas import tpu as pltpu

_BT = 256


def _moe_dense_body(x_ref, wg_ref, w1_ref, w3_ref, w2_ref, out_ref):
    e = pl.program_id(1)
    x = x_ref[...]
    logits = jnp.dot(x, wg_ref[...], preferred_element_type=jnp.float32)
    probs = jax.nn.softmax(logits, axis=-1)
    m1 = jnp.max(probs, axis=-1)
    i1 = jnp.argmax(probs, axis=-1)
    lane = jax.lax.broadcasted_iota(jnp.int32, probs.shape, 1)
    probs2 = jnp.where(lane == i1[:, None], -jnp.inf, probs)
    m2 = jnp.max(probs2, axis=-1)
    i2 = jnp.argmax(probs2, axis=-1)
    w = jnp.where(e == i1, m1, jnp.where(e == i2, m2, 0.0)) / (m1 + m2)
    xb = x.astype(jnp.bfloat16)
    g = jnp.dot(xb, w1_ref[0], preferred_element_type=jnp.float32)
    u = jnp.dot(xb, w3_ref[0], preferred_element_type=jnp.float32)
    h = (g * jax.nn.sigmoid(g)) * u
    oe = jnp.dot(h.astype(jnp.bfloat16), w2_ref[0], preferred_element_type=jnp.float32)
    contrib = oe * w[:, None]

    @pl.when(e == 0)
    def _():
        out_ref[...] = contrib

    @pl.when(e != 0)
    def _():
        out_ref[...] += contrib


def kernel(hidden_states, Wg, W1, W3, W2):
    T, D = hidden_states.shape
    E = Wg.shape[1]
    F = W1.shape[2]
    W1 = W1.astype(jnp.bfloat16)
    W3 = W3.astype(jnp.bfloat16)
    W2 = W2.astype(jnp.bfloat16)
    grid = (T // _BT, E)
    return pl.pallas_call(
        _moe_dense_body,
        grid=grid,
        in_specs=[
            pl.BlockSpec((_BT, D), lambda t, e: (t, 0)),
            pl.BlockSpec((D, E), lambda t, e: (0, 0)),
            pl.BlockSpec((1, D, F), lambda t, e: (e, 0, 0)),
            pl.BlockSpec((1, D, F), lambda t, e: (e, 0, 0)),
            pl.BlockSpec((1, F, D), lambda t, e: (e, 0, 0)),
        ],
        out_specs=pl.BlockSpec((_BT, D), lambda t, e: (t, 0)),
        out_shape=jax.ShapeDtypeStruct((T, D), jnp.float32),
        compiler_params=pltpu.CompilerParams(
            dimension_semantics=("parallel", "arbitrary"),
        ),
    )(hidden_states, Wg, W1, W3, W2)


# speedup vs baseline: 1.8489x; 1.5481x over previous
import functools

import jax
import jax.numpy as jnp
from jax.experimental import pallas as pl
from jax.experimental.pallas import tpu as pltpu

_BT = 2048


def _moe_dense_body(x_ref, wg_ref, w1_ref, w3_ref, w2_ref, out_ref,
                    comb_scr, xb_scr):
    e = pl.program_id(1)

    @pl.when(e == 0)
    def _():
        x = x_ref[...]
        logits = jnp.dot(x, wg_ref[...], preferred_element_type=jnp.float32)
        probs = jax.nn.softmax(logits, axis=-1)
        m1 = jnp.max(probs, axis=-1)
        i1 = jnp.argmax(probs, axis=-1)
        lane = jax.lax.broadcasted_iota(jnp.int32, probs.shape, 1)
        probs2 = jnp.where(lane == i1[:, None], -jnp.inf, probs)
        m2 = jnp.max(probs2, axis=-1)
        i2 = jnp.argmax(probs2, axis=-1)
        comb = jnp.where(lane == i1[:, None], m1[:, None],
                         jnp.where(lane == i2[:, None], m2[:, None], 0.0))
        comb_scr[...] = comb / (m1 + m2)[:, None]
        xb_scr[...] = x.astype(jnp.bfloat16)

    xb = xb_scr[...]
    g = jnp.dot(xb, w1_ref[0], preferred_element_type=jnp.float32)
    u = jnp.dot(xb, w3_ref[0], preferred_element_type=jnp.float32)
    h = (g * jax.nn.sigmoid(g)) * u
    oe = jnp.dot(h.astype(jnp.bfloat16), w2_ref[0],
                 preferred_element_type=jnp.float32)
    lane = jax.lax.broadcasted_iota(jnp.int32, comb_scr.shape, 1)
    w = jnp.sum(jnp.where(lane == e, comb_scr[...], 0.0), axis=1)
    contrib = oe * w[:, None]

    @pl.when(e == 0)
    def _():
        out_ref[...] = contrib

    @pl.when(e != 0)
    def _():
        out_ref[...] += contrib


def kernel(hidden_states, Wg, W1, W3, W2):
    T, D = hidden_states.shape
    E = Wg.shape[1]
    F = W1.shape[2]
    W1 = W1.astype(jnp.bfloat16)
    W3 = W3.astype(jnp.bfloat16)
    W2 = W2.astype(jnp.bfloat16)
    grid = (T // _BT, E)
    return pl.pallas_call(
        _moe_dense_body,
        grid=grid,
        in_specs=[
            pl.BlockSpec((_BT, D), lambda t, e: (t, 0)),
            pl.BlockSpec((D, E), lambda t, e: (0, 0)),
            pl.BlockSpec((1, D, F), lambda t, e: (e, 0, 0)),
            pl.BlockSpec((1, D, F), lambda t, e: (e, 0, 0)),
            pl.BlockSpec((1, F, D), lambda t, e: (e, 0, 0)),
        ],
        out_specs=pl.BlockSpec((_BT, D), lambda t, e: (t, 0)),
        out_shape=jax.ShapeDtypeStruct((T, D), jnp.float32),
        scratch_shapes=[
            pltpu.VMEM((_BT, E), jnp.float32),
            pltpu.VMEM((_BT, D), jnp.bfloat16),
        ],
        compiler_params=pltpu.CompilerParams(
            dimension_semantics=("parallel", "arbitrary"),
        ),
    )(hidden_states, Wg, W1, W3, W2)
